# Initial kernel scaffold; baseline (speedup 1.0000x reference)
import jax
import jax.numpy as jnp
from jax.experimental import pallas as pl


def kernel(flat_tokens, gate_weight, noise_weight):
    raise NotImplementedError("write your pallas kernel here")


# speedup vs baseline: 4.8345x; 4.8345x over previous
import functools

import jax
import jax.numpy as jnp
from jax.experimental import pallas as pl
from jax.experimental.pallas import tpu as pltpu

_IN_DIM = 4096
_N_EXPERTS = 64
_TOP_K = 8
_N_TOKENS = 16384
_ROWS = 512


def _cv2(v):
    n = v.shape[-1]
    mean = jnp.sum(v) / n
    var = jnp.sum((v - mean) ** 2) / (n - 1)
    return var / (mean * mean + 1e-10)


def _router_body(x_ref, w_ref, logits_ref, gates_ref, imp_ref, load_ref,
                 loss_ref, zsum_ref):
    i = pl.program_id(0)
    nsteps = pl.num_programs(0)

    @pl.when(i == 0)
    def _init():
        imp_ref[:] = jnp.zeros_like(imp_ref)
        load_ref[:] = jnp.zeros_like(load_ref)
        zsum_ref[0, 0] = jnp.float32(0.0)

    logits = jnp.dot(x_ref[:], w_ref[:], preferred_element_type=jnp.float32)
    logits_ref[:] = logits

    lane = jax.lax.broadcasted_iota(jnp.int32, logits.shape, 1)
    work = logits
    mask = jnp.zeros(logits.shape, dtype=jnp.bool_)
    neg = jnp.float32(-jnp.inf)
    for _ in range(_TOP_K):
        m = jnp.max(work, axis=1, keepdims=True)
        eq = work == m
        idx = jnp.min(jnp.where(eq, lane, _N_EXPERTS), axis=1, keepdims=True)
        sel = lane == idx
        mask = jnp.logical_or(mask, sel)
        work = jnp.where(sel, neg, work)

    rowmax = jnp.max(logits, axis=1, keepdims=True)
    e = jnp.where(mask, jnp.exp(logits - rowmax), jnp.float32(0.0))
    denom = jnp.sum(e, axis=1, keepdims=True)
    gates = e / denom
    gates_ref[:] = gates

    imp_ref[:] += jnp.sum(gates, axis=0, keepdims=True)
    load_ref[:] += jnp.sum((gates > 0).astype(jnp.int32), axis=0,
                           keepdims=True)
    lse = jnp.log(jnp.sum(jnp.exp(logits), axis=1))
    zsum_ref[0, 0] += jnp.sum(lse)

    @pl.when(i == nsteps - 1)
    def _finish():
        imp = imp_ref[:].reshape(_N_EXPERTS)
        load = load_ref[:].astype(jnp.float32).reshape(_N_EXPERTS)
        z = zsum_ref[0, 0] / jnp.float32(_N_TOKENS)
        loss_ref[0, 0] = _cv2(imp) + _cv2(load) + z


@jax.jit
def kernel(flat_tokens, gate_weight, noise_weight):
    del noise_weight
    n_tokens = flat_tokens.shape[0]
    grid = (n_tokens // _ROWS,)
    out_shape = (
        jax.ShapeDtypeStruct((n_tokens, _N_EXPERTS), jnp.float32),
        jax.ShapeDtypeStruct((n_tokens, _N_EXPERTS), jnp.float32),
        jax.ShapeDtypeStruct((1, _N_EXPERTS), jnp.float32),
        jax.ShapeDtypeStruct((1, _N_EXPERTS), jnp.int32),
        jax.ShapeDtypeStruct((1, 1), jnp.float32),
    )
    in_specs = [
        pl.BlockSpec((_ROWS, _IN_DIM), lambda i: (i, 0)),
        pl.BlockSpec((_IN_DIM, _N_EXPERTS), lambda i: (0, 0)),
    ]
    out_specs = (
        pl.BlockSpec((_ROWS, _N_EXPERTS), lambda i: (i, 0)),
        pl.BlockSpec((_ROWS, _N_EXPERTS), lambda i: (i, 0)),
        pl.BlockSpec((1, _N_EXPERTS), lambda i: (0, 0)),
        pl.BlockSpec((1, _N_EXPERTS), lambda i: (0, 0)),
        pl.BlockSpec(memory_space=pltpu.SMEM),
    )
    logits, gates, imp, load, loss = pl.pallas_call(
        _router_body,
        grid=grid,
        in_specs=in_specs,
        out_specs=out_specs,
        out_shape=out_shape,
        scratch_shapes=[pltpu.SMEM((1, 1), jnp.float32)],
    )(flat_tokens, gate_weight)
    return (gates, load.reshape(_N_EXPERTS), logits, loss[0, 0],
            imp.reshape(_N_EXPERTS))


# speedup vs baseline: 5.8370x; 1.2074x over previous
import functools

import jax
import jax.numpy as jnp
from jax.experimental import pallas as pl
from jax.experimental.pallas import tpu as pltpu

_IN_DIM = 4096
_N_EXPERTS = 64
_TOP_K = 8
_N_TOKENS = 16384
_ROWS = 512


def _cv2(v):
    n = v.shape[-1]
    mean = jnp.sum(v) / n
    var = jnp.sum((v - mean) ** 2) / (n - 1)
    return var / (mean * mean + 1e-10)


def _router_body(x_ref, w_ref, logits_ref, gates_ref, imp_ref, load_ref,
                 loss_ref, zsum_ref):
    i = pl.program_id(0)
    nsteps = pl.num_programs(0)

    @pl.when(i == 0)
    def _init():
        imp_ref[:] = jnp.zeros_like(imp_ref)
        load_ref[:] = jnp.zeros_like(load_ref)
        zsum_ref[0, 0] = jnp.float32(0.0)

    logits = jnp.dot(x_ref[:], w_ref[:], preferred_element_type=jnp.float32)
    logits_ref[:] = logits

    neg = jnp.float32(-jnp.inf)
    work = logits
    rowmax = jnp.max(work, axis=1, keepdims=True)
    m = rowmax
    for _ in range(_TOP_K):
        work = jnp.where(work == m, neg, work)
        m = jnp.max(work, axis=1, keepdims=True)

    e_all = jnp.exp(logits - rowmax)
    e = jnp.where(work == logits, jnp.float32(0.0), e_all)
    denom = jnp.sum(e, axis=1, keepdims=True)
    gates = e / denom
    gates_ref[:] = gates

    imp_ref[:] += jnp.sum(gates, axis=0, keepdims=True)
    load_ref[:] += jnp.sum((gates > 0).astype(jnp.int32), axis=0,
                           keepdims=True)
    lse = rowmax[:, 0] + jnp.log(jnp.sum(e_all, axis=1))
    zsum_ref[0, 0] += jnp.sum(lse)

    @pl.when(i == nsteps - 1)
    def _finish():
        imp = imp_ref[:].reshape(_N_EXPERTS)
        load = load_ref[:].astype(jnp.float32).reshape(_N_EXPERTS)
        z = zsum_ref[0, 0] / jnp.float32(_N_TOKENS)
        loss_ref[0, 0] = _cv2(imp) + _cv2(load) + z


@jax.jit
def kernel(flat_tokens, gate_weight, noise_weight):
    del noise_weight
    n_tokens = flat_tokens.shape[0]
    grid = (n_tokens // _ROWS,)
    out_shape = (
        jax.ShapeDtypeStruct((n_tokens, _N_EXPERTS), jnp.float32),
        jax.ShapeDtypeStruct((n_tokens, _N_EXPERTS), jnp.float32),
        jax.ShapeDtypeStruct((1, _N_EXPERTS), jnp.float32),
        jax.ShapeDtypeStruct((1, _N_EXPERTS), jnp.int32),
        jax.ShapeDtypeStruct((1, 1), jnp.float32),
    )
    in_specs = [
        pl.BlockSpec((_ROWS, _IN_DIM), lambda i: (i, 0)),
        pl.BlockSpec((_IN_DIM, _N_EXPERTS), lambda i: (0, 0)),
    ]
    out_specs = (
        pl.BlockSpec((_ROWS, _N_EXPERTS), lambda i: (i, 0)),
        pl.BlockSpec((_ROWS, _N_EXPERTS), lambda i: (i, 0)),
        pl.BlockSpec((1, _N_EXPERTS), lambda i: (0, 0)),
        pl.BlockSpec((1, _N_EXPERTS), lambda i: (0, 0)),
        pl.BlockSpec(memory_space=pltpu.SMEM),
    )
    logits, gates, imp, load, loss = pl.pallas_call(
        _router_body,
        grid=grid,
        in_specs=in_specs,
        out_specs=out_specs,
        out_shape=out_shape,
        scratch_shapes=[pltpu.SMEM((1, 1), jnp.float32)],
    )(flat_tokens, gate_weight)
    return (gates, load.reshape(_N_EXPERTS), logits, loss[0, 0],
            imp.reshape(_N_EXPERTS))


# speedup vs baseline: 6.2869x; 1.0771x over previous
import functools

import jax
import jax.numpy as jnp
from jax.experimental import pallas as pl
from jax.experimental.pallas import tpu as pltpu

_IN_DIM = 4096
_N_EXPERTS = 64
_TOP_K = 8
_N_TOKENS = 16384
_ROWS = 1024


def _cv2(v):
    n = v.shape[-1]
    mean = jnp.sum(v) / n
    var = jnp.sum((v - mean) ** 2) / (n - 1)
    return var / (mean * mean + 1e-10)


def _router_body(x_ref, w_ref, logits_ref, gates_ref, imp_ref, load_ref,
                 loss_ref, zsum_ref):
    i = pl.program_id(0)
    nsteps = pl.num_programs(0)

    @pl.when(i == 0)
    def _init():
        imp_ref[:] = jnp.zeros_like(imp_ref)
        load_ref[:] = jnp.zeros_like(load_ref)
        zsum_ref[0, 0] = jnp.float32(0.0)

    logits = jnp.dot(x_ref[:], w_ref[:], preferred_element_type=jnp.float32)
    logits_ref[:] = logits

    neg = jnp.float32(-jnp.inf)
    work = logits
    rowmax = jnp.max(work, axis=1, keepdims=True)
    m = rowmax
    for _ in range(_TOP_K):
        work = jnp.where(work == m, neg, work)
        m = jnp.max(work, axis=1, keepdims=True)

    e_all = jnp.exp(logits - rowmax)
    e = jnp.where(work == logits, jnp.float32(0.0), e_all)
    denom = jnp.sum(e, axis=1, keepdims=True)
    gates = e / denom
    gates_ref[:] = gates

    imp_ref[:] += jnp.sum(gates, axis=0, keepdims=True)
    load_ref[:] += jnp.sum((gates > 0).astype(jnp.int32), axis=0,
                           keepdims=True)
    lse = rowmax[:, 0] + jnp.log(jnp.sum(e_all, axis=1))
    zsum_ref[0, 0] += jnp.sum(lse)

    @pl.when(i == nsteps - 1)
    def _finish():
        imp = imp_ref[:].reshape(_N_EXPERTS)
        load = load_ref[:].astype(jnp.float32).reshape(_N_EXPERTS)
        z = zsum_ref[0, 0] / jnp.float32(_N_TOKENS)
        loss_ref[0, 0] = _cv2(imp) + _cv2(load) + z


@jax.jit
def kernel(flat_tokens, gate_weight, noise_weight):
    del noise_weight
    n_tokens = flat_tokens.shape[0]
    grid = (n_tokens // _ROWS,)
    out_shape = (
        jax.ShapeDtypeStruct((n_tokens, _N_EXPERTS), jnp.float32),
        jax.ShapeDtypeStruct((n_tokens, _N_EXPERTS), jnp.float32),
        jax.ShapeDtypeStruct((1, _N_EXPERTS), jnp.float32),
        jax.ShapeDtypeStruct((1, _N_EXPERTS), jnp.int32),
        jax.ShapeDtypeStruct((1, 1), jnp.float32),
    )
    in_specs = [
        pl.BlockSpec((_ROWS, _IN_DIM), lambda i: (i, 0)),
        pl.BlockSpec((_IN_DIM, _N_EXPERTS), lambda i: (0, 0)),
    ]
    out_specs = (
        pl.BlockSpec((_ROWS, _N_EXPERTS), lambda i: (i, 0)),
        pl.BlockSpec((_ROWS, _N_EXPERTS), lambda i: (i, 0)),
        pl.BlockSpec((1, _N_EXPERTS), lambda i: (0, 0)),
        pl.BlockSpec((1, _N_EXPERTS), lambda i: (0, 0)),
        pl.BlockSpec(memory_space=pltpu.SMEM),
    )
    logits, gates, imp, load, loss = pl.pallas_call(
        _router_body,
        grid=grid,
        in_specs=in_specs,
        out_specs=out_specs,
        out_shape=out_shape,
        scratch_shapes=[pltpu.SMEM((1, 1), jnp.float32)],
    )(flat_tokens, gate_weight)
    return (gates, load.reshape(_N_EXPERTS), logits, loss[0, 0],
            imp.reshape(_N_EXPERTS))
